# Initial kernel scaffold; baseline (speedup 1.0000x reference)
import jax
import jax.numpy as jnp
from jax.experimental import pallas as pl


def kernel(cluster, q, Wq, Wkv, Wp, bp):
    raise NotImplementedError("write your pallas kernel here")


# speedup vs baseline: 4.8508x; 4.8508x over previous
import jax
import jax.numpy as jnp
from jax.experimental import pallas as pl
from jax.experimental.pallas import tpu as pltpu

H = 16
C = 128


def _proj_kernel(q_ref, w_ref, qh_ref, kv_ref):
    x = q_ref[...].astype(jnp.bfloat16)
    y = jax.lax.dot_general(
        x, w_ref[...], (((1,), (1,)), ((), ())),
        preferred_element_type=jnp.float32)
    d = qh_ref.shape[1]
    qh_ref[...] = y[:, :d].astype(jnp.bfloat16)
    kv_ref[...] = y[:, d:].astype(jnp.bfloat16)


def _center_kernel(cl_ref, kv_ref, cent_ref, cnt_ref):
    cl = cl_ref[0]
    n = cl.shape[1]
    iota = jax.lax.broadcasted_iota(jnp.int32, (C, n), 0)
    oh = (iota == cl).astype(jnp.bfloat16)
    cent_ref[0] = jax.lax.dot_general(
        oh, kv_ref[0], (((1,), (0,)), ((), ())),
        preferred_element_type=jnp.float32).astype(jnp.bfloat16)
    cnt_ref[0] = jnp.sum(oh.astype(jnp.float32), axis=1).reshape(1, C)


def _attn_kernel(qh_ref, cent_ref, cnt_ref, out_ref):
    qh = qh_ref[0]
    n, d = qh.shape
    dh = d // H
    nb = cent_ref.shape[1]
    for h in range(H):
        r = h % nb
        qh_h = qh[:, h * dh:(h + 1) * dh]
        kc = cent_ref[0, r, :, h * dh:(h + 1) * dh]
        vc = cent_ref[0, r, :, d + h * dh:d + (h + 1) * dh]
        cnt = cnt_ref[0, r]
        w = jnp.where(cnt > 0, 1.0 / cnt, 0.0)
        ind = (cnt > 0).astype(jnp.float32)
        s = jax.lax.dot_general(
            qh_h, kc, (((1,), (1,)), ((), ())),
            preferred_element_type=jnp.float32)
        s = s * (w * jax.lax.rsqrt(jnp.float32(dh)))
        m = jnp.max(s, axis=1, keepdims=True)
        e = jnp.exp(s - m)
        denom = jnp.sum(e * cnt, axis=1, keepdims=True)
        num = jax.lax.dot_general(
            (e * ind).astype(jnp.bfloat16), vc, (((1,), (0,)), ((), ())),
            preferred_element_type=jnp.float32)
        out_ref[0, :, h * dh:(h + 1) * dh] = (num / denom).astype(jnp.bfloat16)


def _outproj_kernel(x_ref, w_ref, b_ref, o_ref):
    y = jax.lax.dot_general(
        x_ref[...], w_ref[...], (((1,), (1,)), ((), ())),
        preferred_element_type=jnp.float32)
    o_ref[...] = y + b_ref[...]


def kernel(cluster, q, Wq, Wkv, Wp, bp):
    B, N, D = q.shape
    dh = D // H
    q2 = q.reshape(B * N, D)
    w_cat = jnp.concatenate([Wq, Wkv], axis=0).astype(jnp.bfloat16)

    mblk = 512
    qh2, kv2 = pl.pallas_call(
        _proj_kernel,
        grid=(B * N // mblk,),
        in_specs=[
            pl.BlockSpec((mblk, D), lambda i: (i, 0)),
            pl.BlockSpec((3 * D, D), lambda i: (0, 0)),
        ],
        out_specs=[
            pl.BlockSpec((mblk, D), lambda i: (i, 0)),
            pl.BlockSpec((mblk, 2 * D), lambda i: (i, 0)),
        ],
        out_shape=[
            jax.ShapeDtypeStruct((B * N, D), jnp.bfloat16),
            jax.ShapeDtypeStruct((B * N, 2 * D), jnp.bfloat16),
        ],
    )(q2, w_cat)

    cl3 = cluster.reshape(B, 1, N)
    kv3 = kv2.reshape(B, N, 2 * D)
    centers, counts = pl.pallas_call(
        _center_kernel,
        grid=(B * B,),
        in_specs=[
            pl.BlockSpec((1, 1, N), lambda i: (i % B, 0, 0)),
            pl.BlockSpec((1, N, 2 * D), lambda i: (i // B, 0, 0)),
        ],
        out_specs=[
            pl.BlockSpec((1, C, 2 * D), lambda i: (i, 0, 0)),
            pl.BlockSpec((1, 1, C), lambda i: (i, 0, 0)),
        ],
        out_shape=[
            jax.ShapeDtypeStruct((B * B, C, 2 * D), jnp.bfloat16),
            jax.ShapeDtypeStruct((B * B, 1, C), jnp.float32),
        ],
    )(cl3, kv3)

    qh3 = qh2.reshape(B, N, D)
    cent4 = centers.reshape(B, B, C, 2 * D)
    cnt4 = counts.reshape(B, B, 1, C)
    out_heads = pl.pallas_call(
        _attn_kernel,
        grid=(B,),
        in_specs=[
            pl.BlockSpec((1, N, D), lambda b: (b, 0, 0)),
            pl.BlockSpec((1, B, C, 2 * D), lambda b: (b, 0, 0, 0)),
            pl.BlockSpec((1, B, 1, C), lambda b: (b, 0, 0, 0)),
        ],
        out_specs=pl.BlockSpec((1, N, D), lambda b: (b, 0, 0)),
        out_shape=jax.ShapeDtypeStruct((B, N, D), jnp.bfloat16),
    )(qh3, cent4, cnt4)

    x2 = out_heads.reshape(B * N, D)
    bp2 = bp.reshape(1, D)
    out = pl.pallas_call(
        _outproj_kernel,
        grid=(B * N // mblk,),
        in_specs=[
            pl.BlockSpec((mblk, D), lambda i: (i, 0)),
            pl.BlockSpec((D, D), lambda i: (0, 0)),
            pl.BlockSpec((1, D), lambda i: (0, 0)),
        ],
        out_specs=pl.BlockSpec((mblk, D), lambda i: (i, 0)),
        out_shape=jax.ShapeDtypeStruct((B * N, D), jnp.float32),
    )(x2, Wp.astype(jnp.bfloat16), bp2)

    return out.reshape(B, N, D)


# speedup vs baseline: 5.7686x; 1.1892x over previous
import jax
import jax.numpy as jnp
from jax.experimental import pallas as pl
from jax.experimental.pallas import tpu as pltpu

H = 16
C = 128


def _mega_kernel(cl_ref, q_ref, wq_ref, wkv_ref, wp_ref, bp_ref, out_ref):
    n, d = q_ref.shape[1], q_ref.shape[2]
    nb = cl_ref.shape[0]
    dh = d // H
    x = q_ref[0]
    qh = jax.lax.dot_general(
        x, wq_ref[...], (((1,), (1,)), ((), ())),
        preferred_element_type=jnp.float32).astype(jnp.bfloat16)

    iota = jax.lax.broadcasted_iota(jnp.int32, (C, n), 0)
    log2e = 1.4426950408889634
    ohs = [(iota == cl_ref[r]).astype(jnp.bfloat16) for r in range(nb)]
    oh_all = jnp.concatenate(ohs, axis=0)
    cnt_col = jnp.sum(oh_all.astype(jnp.float32), axis=1, keepdims=True)
    xs = jax.lax.dot_general(
        oh_all, x, (((1,), (0,)), ((), ())),
        preferred_element_type=jnp.float32).astype(jnp.bfloat16)
    cents = jax.lax.dot_general(
        xs, wkv_ref[...], (((1,), (1,)), ((), ())),
        preferred_element_type=jnp.float32)

    w_col = jnp.where(cnt_col > 0, 1.0 / cnt_col, 0.0)
    a_col = w_col * (log2e * jax.lax.rsqrt(jnp.float32(dh)))
    lc_rows = [jnp.log2(cnt_col[r * C:(r + 1) * C]).reshape(1, C)
               for r in range(nb)]

    outs = []
    for h in range(H):
        r = h % nb
        rs = slice(r * C, (r + 1) * C)
        qh_h = qh[:, h * dh:(h + 1) * dh]
        kc = (cents[rs, h * dh:(h + 1) * dh]
              * a_col[rs]).astype(jnp.bfloat16)
        vc = (cents[rs, d + h * dh:d + (h + 1) * dh]
              * w_col[rs]).astype(jnp.bfloat16)
        t = jax.lax.dot_general(
            qh_h, kc, (((1,), (1,)), ((), ())),
            preferred_element_type=jnp.float32) + lc_rows[r]
        m = jnp.max(t, axis=1, keepdims=True)
        e = jnp.exp2(t - m)
        denom = jnp.sum(e, axis=1, keepdims=True)
        num = jax.lax.dot_general(
            e.astype(jnp.bfloat16), vc, (((1,), (0,)), ((), ())),
            preferred_element_type=jnp.float32)
        outs.append((num * (1.0 / denom)).astype(jnp.bfloat16))
    ao = jnp.concatenate(outs, axis=1)

    out_ref[0] = jax.lax.dot_general(
        ao, wp_ref[...], (((1,), (1,)), ((), ())),
        preferred_element_type=jnp.float32) + bp_ref[...]


def kernel(cluster, q, Wq, Wkv, Wp, bp):
    B, N, D = q.shape
    cl3 = cluster.reshape(B, 1, N)
    bp2 = bp.reshape(1, D)

    out = pl.pallas_call(
        _mega_kernel,
        grid=(B,),
        in_specs=[
            pl.BlockSpec((B, 1, N), lambda b: (0, 0, 0)),
            pl.BlockSpec((1, N, D), lambda b: (b, 0, 0)),
            pl.BlockSpec((D, D), lambda b: (0, 0)),
            pl.BlockSpec((2 * D, D), lambda b: (0, 0)),
            pl.BlockSpec((D, D), lambda b: (0, 0)),
            pl.BlockSpec((1, D), lambda b: (0, 0)),
        ],
        out_specs=pl.BlockSpec((1, N, D), lambda b: (b, 0, 0)),
        out_shape=jax.ShapeDtypeStruct((B, N, D), jnp.float32),
    )(cl3, q.astype(jnp.bfloat16),
      Wq.astype(jnp.bfloat16), Wkv.astype(jnp.bfloat16),
      Wp.astype(jnp.bfloat16), bp2)

    return out
